# baseline (speedup 1.0000x reference)
import jax
import jax.numpy as jnp
from jax.experimental import pallas as pl
from jax.experimental.pallas import tpu as pltpu

N_WIN2 = 49
TOPK = 4
C_CHUNK = 128


def _router_kernel(x_ref, w_ref, b_ref, mask_ref, rs_ref, *, nc):
    c = pl.program_id(1)
    part = jax.lax.dot_general(
        x_ref[0], w_ref[...], (((0,), (1,)), ((), ())),
        preferred_element_type=jnp.float32)

    @pl.when(c == 0)
    def _():
        rs_ref[0] = part

    @pl.when(c != 0)
    def _():
        rs_ref[0] += part

    @pl.when(c == nc - 1)
    def _():
        s = rs_ref[0] + b_ref[0][None, :]
        m = jnp.max(s, axis=-1, keepdims=True)
        e = jnp.exp(s - m)
        rs_ref[0] = e / jnp.sum(e, axis=-1, keepdims=True)
        work = s
        for _ in range(TOPK - 1):
            mx = jnp.max(work, axis=-1, keepdims=True)
            work = jnp.where(work == mx, -jnp.inf, work)
        t4 = jnp.max(work, axis=-1, keepdims=True)
        mask_ref[0] = jnp.where(s >= t4, 1.0, 0.0)


def kernel(x, W, b):
    import functools
    B, C, H, Wd = x.shape
    HW = H * Wd
    nc = C // C_CHUNK
    x3 = x.reshape(B, C, HW)
    b2 = b.reshape(1, N_WIN2)
    out_shape = [
        jax.ShapeDtypeStruct((B, HW, N_WIN2), jnp.float32),
        jax.ShapeDtypeStruct((B, HW, N_WIN2), jnp.float32),
    ]
    mask, rs = pl.pallas_call(
        functools.partial(_router_kernel, nc=nc),
        grid=(B, nc),
        in_specs=[
            pl.BlockSpec((1, C_CHUNK, HW), lambda bb, c: (bb, c, 0)),
            pl.BlockSpec((N_WIN2, C_CHUNK), lambda bb, c: (0, c)),
            pl.BlockSpec((1, N_WIN2), lambda bb, c: (0, 0)),
        ],
        out_specs=[
            pl.BlockSpec((1, HW, N_WIN2), lambda bb, c: (bb, 0, 0)),
            pl.BlockSpec((1, HW, N_WIN2), lambda bb, c: (bb, 0, 0)),
        ],
        out_shape=out_shape,
    )(x3, W, b2)
    return (mask, rs)


# speedup vs baseline: 1.3069x; 1.3069x over previous
import jax
import jax.numpy as jnp
from jax.experimental import pallas as pl
from jax.experimental.pallas import tpu as pltpu

N_WIN2 = 49
TOPK = 4


def _router_kernel(x_ref, w_ref, b_ref, mask_ref, rs_ref):
    s = jax.lax.dot_general(
        w_ref[...], x_ref[0], (((1,), (0,)), ((), ())),
        preferred_element_type=jnp.float32)
    s = s + b_ref[...]

    m = jnp.max(s, axis=0, keepdims=True)
    e = jnp.exp(s - m)
    r = e / jnp.sum(e, axis=0, keepdims=True)

    work = s
    for _ in range(TOPK - 1):
        mx = jnp.max(work, axis=0, keepdims=True)
        work = jnp.where(work == mx, -jnp.inf, work)
    t4 = jnp.max(work, axis=0, keepdims=True)
    msk = jnp.where(s >= t4, 1.0, 0.0)

    i0 = jax.lax.broadcasted_iota(jnp.int32, (N_WIN2, N_WIN2), 0)
    i1 = jax.lax.broadcasted_iota(jnp.int32, (N_WIN2, N_WIN2), 1)
    eye = jnp.where(i0 == i1, 1.0, 0.0)
    rs_ref[0] = jax.lax.dot_general(
        r, eye, (((0,), (0,)), ((), ())), preferred_element_type=jnp.float32)
    mask_ref[0] = jax.lax.dot_general(
        msk, eye, (((0,), (0,)), ((), ())), preferred_element_type=jnp.float32)


def kernel(x, W, b):
    B, C, H, Wd = x.shape
    HW = H * Wd
    x3 = x.reshape(B, C, HW)
    b2 = b.reshape(N_WIN2, 1)
    out_shape = [
        jax.ShapeDtypeStruct((B, HW, N_WIN2), jnp.float32),
        jax.ShapeDtypeStruct((B, HW, N_WIN2), jnp.float32),
    ]
    mask, rs = pl.pallas_call(
        _router_kernel,
        grid=(B,),
        in_specs=[
            pl.BlockSpec((1, C, HW), lambda bb: (bb, 0, 0)),
            pl.BlockSpec((N_WIN2, C), lambda bb: (0, 0)),
            pl.BlockSpec((N_WIN2, 1), lambda bb: (0, 0)),
        ],
        out_specs=[
            pl.BlockSpec((1, HW, N_WIN2), lambda bb: (bb, 0, 0)),
            pl.BlockSpec((1, HW, N_WIN2), lambda bb: (bb, 0, 0)),
        ],
        out_shape=out_shape,
    )(x3, W, b2)
    return (mask, rs)
